# baseline (speedup 1.0000x reference)
import functools

import jax
import jax.numpy as jnp
from jax import lax
from jax.experimental import pallas as pl
from jax.experimental.pallas import tpu as pltpu
from jax.experimental.pallas import tpu_sc as plsc

_B, _Cin, _C, _H, _W = 2, 64, 64, 224, 224
_HW = _H * _W
_NS, _K = 12544, 9
_HID = (2 * _C + 3) // 2
_NW = 32
_CH = 112
_NB = 128
_NKB = _NB * _K


def _conv(x, w, b):
    y = lax.conv_general_dilated(x, w, (1, 1), ((1, 1), (1, 1)),
                                 dimension_numbers=('NCHW', 'OIHW', 'NCHW'))
    return y + b[None, :, None, None]


def _conv_to_rows(x, w, b):
    y = lax.conv_general_dilated(x, w, (1, 1), ((1, 1), (1, 1)),
                                 dimension_numbers=('NCHW', 'OIHW', 'NHWC'))
    return y + b[None, None, None, :]


def _conv_from_rows(x, w, b):
    y = lax.conv_general_dilated(x, w, (1, 1), ((1, 1), (1, 1)),
                                 dimension_numbers=('NHWC', 'OIHW', 'NCHW'))
    return y + b[None, :, None, None]


_NBUF = 7


@functools.lru_cache(maxsize=None)
def _sc_gather(n_rows, n_idx):
    per_w = n_idx // _NW
    n_chunks = per_w // _CH
    assert per_w % _CH == 0 and n_chunks % _NBUF == 0, (n_rows, n_idx)
    n_groups = n_chunks // _NBUF
    mesh = plsc.VectorSubcoreMesh(core_axis_name="c", subcore_axis_name="s")

    @functools.partial(
        pl.kernel, mesh=mesh,
        out_type=jax.ShapeDtypeStruct((n_idx, _C), jnp.float32),
        scratch_types=[
            [pltpu.VMEM((_CH,), jnp.int32) for _ in range(_NBUF)],
            [pltpu.VMEM((_CH, _C), jnp.float32) for _ in range(_NBUF)],
            pltpu.SemaphoreType.DMA,
        ],
        compiler_params=pltpu.CompilerParams(use_tc_tiling_on_sc=False),
    )
    def k(tab, idx, out, idx_vs, rows_vs, sem):
        wid = lax.axis_index("s") * 2 + lax.axis_index("c")
        base = wid * per_w

        def body(g, c):
            off = base + g * (_CH * _NBUF)
            copies = []
            for j in range(_NBUF):
                pltpu.sync_copy(idx.at[pl.ds(off + j * _CH, _CH)], idx_vs[j])
                copies.append(
                    pltpu.async_copy(tab.at[idx_vs[j]], rows_vs[j], sem))
            for j in range(_NBUF):
                copies[j].wait()
                pltpu.sync_copy(rows_vs[j],
                                out.at[pl.ds(off + j * _CH, _CH)])
            return c

        lax.fori_loop(0, n_groups, body, 0)

    return k


def _attn_body(nn_ref, ds_ref, rs_ref, disp_ref,
               wnn_d_ref, was_d_ref, wbs_d_ref, wcs_d_ref, b1_d_ref, w2_d_ref,
               wnn_r_ref, was_r_ref, wbs_r_ref, wcs_r_ref, b1_r_ref, w2_r_ref,
               dbias_ref, rbias_ref, wd_ref, wr_ref):
    nn = nn_ref[...]
    ds = ds_ref[...]
    rs = rs_ref[...]
    disp = disp_ref[...]
    nn3 = nn.reshape(_NB, _K, _C)
    colmask = lax.broadcasted_iota(jnp.int32, (1, _C), 1) > 0

    def branch(wnn_ref, was_ref, wbs_ref, wcs_ref, b1_ref, w2_ref, bias_ref, sfeat):
        h = jnp.dot(nn, wnn_ref[...], preferred_element_type=jnp.float32)
        h = h + jnp.dot(disp, wcs_ref[...], preferred_element_type=jnp.float32)
        hs = (jnp.dot(ds, was_ref[...], preferred_element_type=jnp.float32)
              + jnp.dot(rs, wbs_ref[...], preferred_element_type=jnp.float32)
              - b1_ref[...])
        h = h - jnp.broadcast_to(hs.reshape(_NB, 1, _HID),
                                 (_NB, _K, _HID)).reshape(_NKB, _HID)
        h = jnp.where(h >= 0, h, 0.2 * h)
        logit = jnp.dot(h, w2_ref[...], preferred_element_type=jnp.float32)
        logit = logit.reshape(_NB, _K)
        m = jnp.max(logit, axis=1, keepdims=True)
        e = jnp.exp(logit - m)
        a = e / jnp.sum(e, axis=1, keepdims=True)
        acc = jnp.zeros((_NB, _C), jnp.float32)
        for kk in range(_K):
            acc = acc + a[:, kk:kk + 1] * nn3[:, kk, :]
        out = acc + bias_ref[...]
        return out + jnp.where(colmask, sfeat, 0.0)

    wd_ref[...] = branch(wnn_d_ref, was_d_ref, wbs_d_ref, wcs_d_ref,
                         b1_d_ref, w2_d_ref, dbias_ref, ds)
    wr_ref[...] = branch(wnn_r_ref, was_r_ref, wbs_r_ref, wcs_r_ref,
                         b1_r_ref, w2_r_ref, rbias_ref, rs)


def _attn_call(nn, ds, rs, disp, wd_args, wr_args, dbias, rbias):
    npts = ds.shape[0]
    grid = npts // _NB
    full = lambda shape: pl.BlockSpec(shape, lambda i: (0, 0))
    in_specs = [
        pl.BlockSpec((_NKB, _C), lambda i: (i, 0)),
        pl.BlockSpec((_NB, _C), lambda i: (i, 0)),
        pl.BlockSpec((_NB, _C), lambda i: (i, 0)),
        pl.BlockSpec((_NKB, 3), lambda i: (i, 0)),
    ]
    for args in (wd_args, wr_args):
        for a in args:
            in_specs.append(full(a.shape))
    in_specs.append(full(dbias.shape))
    in_specs.append(full(rbias.shape))
    out_specs = [pl.BlockSpec((_NB, _C), lambda i: (i, 0)),
                 pl.BlockSpec((_NB, _C), lambda i: (i, 0))]
    return pl.pallas_call(
        _attn_body,
        grid=(grid,),
        in_specs=in_specs,
        out_specs=out_specs,
        out_shape=[jax.ShapeDtypeStruct((npts, _C), jnp.float32),
                   jax.ShapeDtypeStruct((npts, _C), jnp.float32)],
        compiler_params=pltpu.CompilerParams(
            vmem_limit_bytes=100 * 1024 * 1024),
    )(nn, ds, rs, disp, *wd_args, *wr_args, dbias, rbias)


def _split_w1(w1):
    wa = w1[:, :_C]
    wb = w1[:, _C:2 * _C]
    wc = w1[:, 2 * _C:]
    return (wa + wb).T, wa.T, wb.T, wc.T


def kernel(rgb, sdepth, pc_idx, nbrs_idx, nbrs_disp,
           d_w0, d_b0, d_w1, d_b1, d_w2, d_b2,
           r_w0, r_b0, r_w1, r_b1, r_w2, r_b2,
           d_mlp_w1, d_mlp_b1, d_mlp_w2, d_mlp_b2,
           r_mlp_w1, r_mlp_b1, r_mlp_w2, r_mlp_b2,
           d_bias, r_bias):
    d_feat1 = _conv(sdepth, d_w1, d_b1)
    r_feat1 = _conv(rgb, r_w1, r_b1)
    d0T = jax.nn.relu(_conv_to_rows(sdepth, d_w0, d_b0)).reshape(_B, _HW, _C)
    r0T = jax.nn.relu(_conv_to_rows(rgb, r_w0, r_b0)).reshape(_B, _HW, _C)

    pc = pc_idx[:, 0, :].astype(jnp.int32)
    nbr = nbrs_idx[:, 0].astype(jnp.int32)
    boff = (jnp.arange(_B, dtype=jnp.int32) * _HW)[:, None]
    nbr_g = (nbr.reshape(_B, _NS * _K) + boff).reshape(-1)
    pc_g = (pc + boff).reshape(-1)

    idx_d = jnp.concatenate([nbr_g, pc_g])
    out_a = _sc_gather(_B * _HW, idx_d.shape[0])(
        d0T.reshape(_B * _HW, _C), idx_d)
    nn = out_a[:_B * _NS * _K]
    ds = out_a[_B * _NS * _K:]
    rs = _sc_gather(_B * _HW, pc_g.shape[0])(
        r0T.reshape(_B * _HW, _C), pc_g)

    disp = nbrs_disp.transpose(0, 2, 3, 1).reshape(_B * _NS * _K, 3)
    wd_args = (*_split_w1(d_mlp_w1), d_mlp_b1.reshape(1, _HID),
               d_mlp_w2.T)
    wr_args = (*_split_w1(r_mlp_w1), r_mlp_b1.reshape(1, _HID),
               r_mlp_w2.T)
    write_d, write_r = _attn_call(nn, ds, rs, disp, wd_args, wr_args,
                                  d_bias.reshape(1, _C),
                                  r_bias.reshape(1, _C))

    wmap = jnp.full((_B, _HW), -1, jnp.int32).at[
        jnp.arange(_B)[:, None], pc].max(
        jnp.arange(_NS, dtype=jnp.int32)[None, :])

    t_rows = _HW + _NS
    tab = jnp.concatenate([
        d0T[0], write_d[:_NS], d0T[1], write_d[_NS:],
        r0T[0], write_r[:_NS], r0T[1], write_r[_NS:]], axis=0)
    p_ar = jnp.arange(_HW, dtype=jnp.int32)[None, :]
    src_b = jnp.where(wmap >= 0, _HW + wmap, p_ar)
    g_off = (jnp.arange(4, dtype=jnp.int32) * t_rows)[:, None]
    src = (jnp.stack([src_b[0], src_b[1], src_b[0], src_b[1]]) +
           g_off).reshape(-1)
    out_c = _sc_gather(4 * t_rows, src.shape[0])(tab, src)

    newmaps = out_c.reshape(2, _B, _H, _W, _C)
    d_feat2 = _conv_from_rows(newmaps[0], d_w2, d_b2)
    r_feat2 = _conv_from_rows(newmaps[1], r_w2, r_b2)
    return (jax.nn.relu(d_feat2 + d_feat1), jax.nn.relu(r_feat2 + r_feat1))


# speedup vs baseline: 1.0187x; 1.0187x over previous
import functools

import jax
import jax.numpy as jnp
from jax import lax
from jax.experimental import pallas as pl
from jax.experimental.pallas import tpu as pltpu
from jax.experimental.pallas import tpu_sc as plsc

_B, _Cin, _C, _H, _W = 2, 64, 64, 224, 224
_HW = _H * _W
_NS, _K = 12544, 9
_HID = (2 * _C + 3) // 2
_NW = 32
_CH = 112
_NB = 128
_NKB = _NB * _K


def _conv(x, w, b):
    y = lax.conv_general_dilated(x.astype(jnp.bfloat16), w.astype(jnp.bfloat16),
                                 (1, 1), ((1, 1), (1, 1)),
                                 dimension_numbers=('NCHW', 'OIHW', 'NCHW'),
                                 preferred_element_type=jnp.float32)
    return y + b[None, :, None, None]


def _conv_to_rows(x, w, b):
    y = lax.conv_general_dilated(x.astype(jnp.bfloat16), w.astype(jnp.bfloat16),
                                 (1, 1), ((1, 1), (1, 1)),
                                 dimension_numbers=('NCHW', 'OIHW', 'NHWC'),
                                 preferred_element_type=jnp.float32)
    return y + b[None, None, None, :]


def _conv_from_rows(x, w, b):
    y = lax.conv_general_dilated(x.astype(jnp.bfloat16), w.astype(jnp.bfloat16),
                                 (1, 1), ((1, 1), (1, 1)),
                                 dimension_numbers=('NHWC', 'OIHW', 'NCHW'),
                                 preferred_element_type=jnp.float32)
    return y + b[None, :, None, None]


_NBUF = 7


@functools.lru_cache(maxsize=None)
def _sc_gather(n_rows, n_idx):
    per_w = n_idx // _NW
    n_chunks = per_w // _CH
    assert per_w % _CH == 0 and n_chunks % _NBUF == 0, (n_rows, n_idx)
    n_groups = n_chunks // _NBUF
    mesh = plsc.VectorSubcoreMesh(core_axis_name="c", subcore_axis_name="s")

    @functools.partial(
        pl.kernel, mesh=mesh,
        out_type=jax.ShapeDtypeStruct((n_idx, _C), jnp.float32),
        scratch_types=[
            [pltpu.VMEM((_CH,), jnp.int32) for _ in range(_NBUF)],
            [pltpu.VMEM((_CH, _C), jnp.float32) for _ in range(_NBUF)],
            pltpu.SemaphoreType.DMA,
        ],
        compiler_params=pltpu.CompilerParams(use_tc_tiling_on_sc=False),
    )
    def k(tab, idx, out, idx_vs, rows_vs, sem):
        wid = lax.axis_index("s") * 2 + lax.axis_index("c")
        base = wid * per_w

        def body(g, c):
            off = base + g * (_CH * _NBUF)
            copies = []
            for j in range(_NBUF):
                pltpu.sync_copy(idx.at[pl.ds(off + j * _CH, _CH)], idx_vs[j])
                copies.append(
                    pltpu.async_copy(tab.at[idx_vs[j]], rows_vs[j], sem))
            for j in range(_NBUF):
                copies[j].wait()
                pltpu.sync_copy(rows_vs[j],
                                out.at[pl.ds(off + j * _CH, _CH)])
            return c

        lax.fori_loop(0, n_groups, body, 0)

    return k


def _attn_body(nn_ref, ds_ref, rs_ref, disp_ref,
               wnn_d_ref, was_d_ref, wbs_d_ref, wcs_d_ref, b1_d_ref, w2_d_ref,
               wnn_r_ref, was_r_ref, wbs_r_ref, wcs_r_ref, b1_r_ref, w2_r_ref,
               dbias_ref, rbias_ref, wd_ref, wr_ref):
    nn = nn_ref[...]
    ds = ds_ref[...]
    rs = rs_ref[...]
    disp = disp_ref[...]
    nn3 = nn.reshape(_NB, _K, _C)
    colmask = lax.broadcasted_iota(jnp.int32, (1, _C), 1) > 0

    def branch(wnn_ref, was_ref, wbs_ref, wcs_ref, b1_ref, w2_ref, bias_ref, sfeat):
        h = jnp.dot(nn, wnn_ref[...], preferred_element_type=jnp.float32)
        h = h + jnp.dot(disp, wcs_ref[...], preferred_element_type=jnp.float32)
        hs = (jnp.dot(ds, was_ref[...], preferred_element_type=jnp.float32)
              + jnp.dot(rs, wbs_ref[...], preferred_element_type=jnp.float32)
              - b1_ref[...])
        h = h - jnp.broadcast_to(hs.reshape(_NB, 1, _HID),
                                 (_NB, _K, _HID)).reshape(_NKB, _HID)
        h = jnp.where(h >= 0, h, 0.2 * h)
        logit = jnp.dot(h, w2_ref[...], preferred_element_type=jnp.float32)
        logit = logit.reshape(_NB, _K)
        m = jnp.max(logit, axis=1, keepdims=True)
        e = jnp.exp(logit - m)
        a = e / jnp.sum(e, axis=1, keepdims=True)
        acc = jnp.zeros((_NB, _C), jnp.float32)
        for kk in range(_K):
            acc = acc + a[:, kk:kk + 1] * nn3[:, kk, :]
        out = acc + bias_ref[...]
        return out + jnp.where(colmask, sfeat, 0.0)

    wd_ref[...] = branch(wnn_d_ref, was_d_ref, wbs_d_ref, wcs_d_ref,
                         b1_d_ref, w2_d_ref, dbias_ref, ds)
    wr_ref[...] = branch(wnn_r_ref, was_r_ref, wbs_r_ref, wcs_r_ref,
                         b1_r_ref, w2_r_ref, rbias_ref, rs)


def _attn_call(nn, ds, rs, disp, wd_args, wr_args, dbias, rbias):
    npts = ds.shape[0]
    grid = npts // _NB
    full = lambda shape: pl.BlockSpec(shape, lambda i: (0, 0))
    in_specs = [
        pl.BlockSpec((_NKB, _C), lambda i: (i, 0)),
        pl.BlockSpec((_NB, _C), lambda i: (i, 0)),
        pl.BlockSpec((_NB, _C), lambda i: (i, 0)),
        pl.BlockSpec((_NKB, 3), lambda i: (i, 0)),
    ]
    for args in (wd_args, wr_args):
        for a in args:
            in_specs.append(full(a.shape))
    in_specs.append(full(dbias.shape))
    in_specs.append(full(rbias.shape))
    out_specs = [pl.BlockSpec((_NB, _C), lambda i: (i, 0)),
                 pl.BlockSpec((_NB, _C), lambda i: (i, 0))]
    return pl.pallas_call(
        _attn_body,
        grid=(grid,),
        in_specs=in_specs,
        out_specs=out_specs,
        out_shape=[jax.ShapeDtypeStruct((npts, _C), jnp.float32),
                   jax.ShapeDtypeStruct((npts, _C), jnp.float32)],
        compiler_params=pltpu.CompilerParams(
            vmem_limit_bytes=100 * 1024 * 1024),
    )(nn, ds, rs, disp, *wd_args, *wr_args, dbias, rbias)


def _split_w1(w1):
    wa = w1[:, :_C]
    wb = w1[:, _C:2 * _C]
    wc = w1[:, 2 * _C:]
    return (wa + wb).T, wa.T, wb.T, wc.T


def kernel(rgb, sdepth, pc_idx, nbrs_idx, nbrs_disp,
           d_w0, d_b0, d_w1, d_b1, d_w2, d_b2,
           r_w0, r_b0, r_w1, r_b1, r_w2, r_b2,
           d_mlp_w1, d_mlp_b1, d_mlp_w2, d_mlp_b2,
           r_mlp_w1, r_mlp_b1, r_mlp_w2, r_mlp_b2,
           d_bias, r_bias):
    d_feat1 = _conv(sdepth, d_w1, d_b1)
    r_feat1 = _conv(rgb, r_w1, r_b1)
    d0T = jax.nn.relu(_conv_to_rows(sdepth, d_w0, d_b0)).reshape(_B, _HW, _C)
    r0T = jax.nn.relu(_conv_to_rows(rgb, r_w0, r_b0)).reshape(_B, _HW, _C)

    pc = pc_idx[:, 0, :].astype(jnp.int32)
    nbr = nbrs_idx[:, 0].astype(jnp.int32)
    boff = (jnp.arange(_B, dtype=jnp.int32) * _HW)[:, None]
    nbr_g = (nbr.reshape(_B, _NS * _K) + boff).reshape(-1)
    pc_g = (pc + boff).reshape(-1)

    idx_d = jnp.concatenate([nbr_g, pc_g])
    out_a = _sc_gather(_B * _HW, idx_d.shape[0])(
        d0T.reshape(_B * _HW, _C), idx_d)
    nn = out_a[:_B * _NS * _K]
    ds = out_a[_B * _NS * _K:]
    rs = _sc_gather(_B * _HW, pc_g.shape[0])(
        r0T.reshape(_B * _HW, _C), pc_g)

    disp = nbrs_disp.transpose(0, 2, 3, 1).reshape(_B * _NS * _K, 3)
    wd_args = (*_split_w1(d_mlp_w1), d_mlp_b1.reshape(1, _HID),
               d_mlp_w2.T)
    wr_args = (*_split_w1(r_mlp_w1), r_mlp_b1.reshape(1, _HID),
               r_mlp_w2.T)
    write_d, write_r = _attn_call(nn, ds, rs, disp, wd_args, wr_args,
                                  d_bias.reshape(1, _C),
                                  r_bias.reshape(1, _C))

    wmap = jnp.full((_B, _HW), -1, jnp.int32).at[
        jnp.arange(_B)[:, None], pc].max(
        jnp.arange(_NS, dtype=jnp.int32)[None, :])

    t_rows = _HW + _NS
    tab = jnp.concatenate([
        d0T[0], write_d[:_NS], d0T[1], write_d[_NS:],
        r0T[0], write_r[:_NS], r0T[1], write_r[_NS:]], axis=0)
    p_ar = jnp.arange(_HW, dtype=jnp.int32)[None, :]
    src_b = jnp.where(wmap >= 0, _HW + wmap, p_ar)
    g_off = (jnp.arange(4, dtype=jnp.int32) * t_rows)[:, None]
    src = (jnp.stack([src_b[0], src_b[1], src_b[0], src_b[1]]) +
           g_off).reshape(-1)
    out_c = _sc_gather(4 * t_rows, src.shape[0])(tab, src)

    newmaps = out_c.reshape(2, _B, _H, _W, _C)
    d_feat2 = _conv_from_rows(newmaps[0], d_w2, d_b2)
    r_feat2 = _conv_from_rows(newmaps[1], r_w2, r_b2)
    return (jax.nn.relu(d_feat2 + d_feat1), jax.nn.relu(r_feat2 + r_feat1))
